# baseline (speedup 1.0000x reference)
import functools

import jax
import jax.numpy as jnp
from jax import lax
from jax.experimental import pallas as pl
from jax.experimental.pallas import tpu as pltpu
from jax.experimental.pallas import tpu_sc as plsc

N = 10000
D = 128
E = 320000
G = 256

NC = 2
NS = 16
NW = NC * NS

CH = 128
CPW = 80
E_PAD = NW * CPW * CH
N_ACC = 10240
SPC = 40
ZB = 32
OUT_STRIPE = 624

P_ROWS = 320
N_POOL_PAD = NW * P_ROWS
G_ACC = G + 8


def _seg_sum_body(h_hbm, src_hbm, dst_hbm, out_hbm,
                  src_v, dst_v, rows_v, zb_v, agg_sh, sem, sem1):
    cid = lax.axis_index("c")
    sid = lax.axis_index("s")
    wid = sid * NC + cid

    def zfill(i, _):
        for j in range(D // 16):
            zb_v[i, pl.ds(j * 16, 16)] = jnp.zeros((16,), jnp.float32)
        return 0
    lax.fori_loop(0, ZB, zfill, 0)

    def zcopy(m, _):
        pltpu.sync_copy(zb_v, agg_sh.at[pl.ds(sid * (N_ACC // NS) + m * ZB, ZB)])
        return 0
    lax.fori_loop(0, N_ACC // NS // ZB, zcopy, 0)

    plsc.subcore_barrier()

    rows0 = rows_v.at[0]
    rows1 = rows_v.at[1]
    for s in range(CPW // SPC):
        base = wid * CPW + s * SPC
        pltpu.sync_copy(src_hbm.at[pl.ds(base, SPC)], src_v)
        pltpu.sync_copy(dst_hbm.at[pl.ds(base, SPC)], dst_v)
        pltpu.async_copy(h_hbm.at[src_v.at[0]], rows0, sem)

        def pair(p, _):
            c0 = 2 * p
            pltpu.async_copy(h_hbm.at[src_v.at[c0 + 1]], rows1, sem1)
            pltpu.make_async_copy(h_hbm.at[src_v.at[c0]], rows0, sem).wait()
            pltpu.sync_copy(rows0, agg_sh.at[dst_v.at[c0]], add=True)
            pltpu.async_copy(h_hbm.at[src_v.at[c0 + 2]], rows0, sem)
            pltpu.make_async_copy(h_hbm.at[src_v.at[c0 + 1]], rows1, sem1).wait()
            pltpu.sync_copy(rows1, agg_sh.at[dst_v.at[c0 + 1]], add=True)
            return 0
        lax.fori_loop(0, SPC // 2 - 1, pair, 0)

        cl = SPC - 2
        pltpu.async_copy(h_hbm.at[src_v.at[cl + 1]], rows1, sem1)
        pltpu.make_async_copy(h_hbm.at[src_v.at[cl]], rows0, sem).wait()
        pltpu.sync_copy(rows0, agg_sh.at[dst_v.at[cl]], add=True)
        pltpu.make_async_copy(h_hbm.at[src_v.at[cl + 1]], rows1, sem1).wait()
        pltpu.sync_copy(rows1, agg_sh.at[dst_v.at[cl + 1]], add=True)

    plsc.subcore_barrier()

    @pl.when(sid < NS - 1)
    def _():
        pltpu.sync_copy(
            agg_sh.at[pl.ds(sid * OUT_STRIPE, OUT_STRIPE)],
            out_hbm.at[cid, pl.ds(sid * OUT_STRIPE, OUT_STRIPE)])

    @pl.when(sid == NS - 1)
    def _():
        last = (NS - 1) * OUT_STRIPE
        pltpu.sync_copy(
            agg_sh.at[pl.ds(last, N - last)],
            out_hbm.at[cid, pl.ds(last, N - last)])


@jax.jit
def _seg_sum(h, src2d, dst2d):
    mesh = plsc.VectorSubcoreMesh(core_axis_name="c", subcore_axis_name="s")
    return pl.kernel(
        _seg_sum_body,
        out_type=jax.ShapeDtypeStruct((NC, N, D), jnp.float32),
        mesh=mesh,
        scratch_types=[
            pltpu.VMEM((SPC, CH), jnp.int32),
            pltpu.VMEM((SPC, CH), jnp.int32),
            pltpu.VMEM((2, CH, D), jnp.float32),
            pltpu.VMEM((ZB, D), jnp.float32),
            pltpu.VMEM_SHARED((N_ACC, D), jnp.float32),
            pltpu.SemaphoreType.DMA,
            pltpu.SemaphoreType.DMA,
        ],
    )(h, src2d, dst2d)


def _mlp_body(h_ref, a_ref, wa_ref, ba_ref, wb_ref, bb_ref, o_ref):
    z = h_ref[...] + a_ref[0] + a_ref[1]
    z = jnp.dot(z.astype(jnp.bfloat16), wa_ref[...].astype(jnp.bfloat16),
                preferred_element_type=jnp.float32) + ba_ref[...]
    z = jnp.where(z > 0, z, jnp.exp(z) - 1.0)
    z = jnp.dot(z.astype(jnp.bfloat16), wb_ref[...].astype(jnp.bfloat16),
                preferred_element_type=jnp.float32) + bb_ref[...]
    o_ref[...] = jnp.where(z > 0, z, jnp.exp(z) - 1.0)


_MLP_BLK = 1000


@jax.jit
def _mlp(h, agg, wa, ba, wb, bb):
    return pl.pallas_call(
        _mlp_body,
        grid=(N // _MLP_BLK,),
        in_specs=[
            pl.BlockSpec((_MLP_BLK, D), lambda i: (i, 0)),
            pl.BlockSpec((NC, _MLP_BLK, D), lambda i: (0, i, 0)),
            pl.BlockSpec((D, D), lambda i: (0, 0)),
            pl.BlockSpec((1, D), lambda i: (0, 0)),
            pl.BlockSpec((D, D), lambda i: (0, 0)),
            pl.BlockSpec((1, D), lambda i: (0, 0)),
        ],
        out_specs=pl.BlockSpec((_MLP_BLK, D), lambda i: (i, 0)),
        out_shape=jax.ShapeDtypeStruct((N, D), jnp.float32),
    )(h, agg, wa, ba.reshape(1, D), wb, bb.reshape(1, D))


def _pool_body(h_hbm, batch_hbm, out_hbm, hv, bv, pool_v, tmp_v, acc_v,
               pool_sh, sem):
    cid = lax.axis_index("c")
    sid = lax.axis_index("s")
    wid = sid * NC + cid

    pltpu.sync_copy(h_hbm.at[pl.ds(wid * P_ROWS, P_ROWS)], hv)
    pltpu.sync_copy(batch_hbm.at[pl.ds(wid * P_ROWS, P_ROWS)], bv)

    ninf = jnp.full((16,), -jnp.inf, jnp.float32)

    def pinit(i, _):
        for j in range(D // 16):
            pool_v[i, pl.ds(j * 16, 16)] = ninf
        return 0
    lax.fori_loop(0, G_ACC, pinit, 0)

    def grp(g, _):
        b16 = bv[pl.ds(g * 16, 16)]
        for k in range(16):
            b = b16[k]
            i = g * 16 + k
            for j in range(D // 16):
                s = pl.ds(j * 16, 16)
                pool_v[b, s] = jnp.maximum(pool_v[b, s], hv[i, s])
        return 0
    lax.fori_loop(0, P_ROWS // 16, grp, 0)

    pltpu.sync_copy(pool_v.at[pl.ds(0, G)], pool_sh.at[sid])
    plsc.subcore_barrier()

    def ainit(i, _):
        for j in range(D // 16):
            acc_v[i, pl.ds(j * 16, 16)] = ninf
        return 0
    lax.fori_loop(0, 16, ainit, 0)

    def merge(t, _):
        pltpu.sync_copy(pool_sh.at[t, pl.ds(sid * 16, 16)], tmp_v)

        def mrow(i, _):
            for j in range(D // 16):
                s = pl.ds(j * 16, 16)
                acc_v[i, s] = jnp.maximum(acc_v[i, s], tmp_v[i, s])
            return 0
        lax.fori_loop(0, 16, mrow, 0)
        return 0
    lax.fori_loop(0, NS, merge, 0)

    pltpu.sync_copy(acc_v, out_hbm.at[cid, pl.ds(sid * 16, 16)])


@jax.jit
def _pool(h_pad, batch_pad):
    mesh = plsc.VectorSubcoreMesh(core_axis_name="c", subcore_axis_name="s")
    return pl.kernel(
        _pool_body,
        out_type=jax.ShapeDtypeStruct((NC, G, D), jnp.float32),
        mesh=mesh,
        scratch_types=[
            pltpu.VMEM((P_ROWS, D), jnp.float32),
            pltpu.VMEM((P_ROWS,), jnp.int32),
            pltpu.VMEM((G_ACC, D), jnp.float32),
            pltpu.VMEM((16, D), jnp.float32),
            pltpu.VMEM((16, D), jnp.float32),
            pltpu.VMEM_SHARED((NS, G, D), jnp.float32),
            pltpu.SemaphoreType.DMA,
        ],
    )(h_pad, batch_pad)


def _final_body(p_ref, w_ref, b_ref, o_ref):
    pooled = jnp.maximum(p_ref[0], p_ref[1])
    o_ref[...] = jnp.dot(pooled.astype(jnp.bfloat16),
                         w_ref[...].astype(jnp.bfloat16),
                         preferred_element_type=jnp.float32) + b_ref[...]


@jax.jit
def _final(p, wlin, blin):
    c = wlin.shape[1]
    return pl.pallas_call(
        _final_body,
        out_shape=jax.ShapeDtypeStruct((G, c), jnp.float32),
    )(p, wlin, blin.reshape(1, c))


def kernel(x, edge_index, batch, Wa, ba, Wb, bb, Wlin, blin):
    src = edge_index[0]
    dst = edge_index[1]
    order = jnp.argsort(dst, stable=True)
    src = src[order]
    dst = dst[order]
    pad = E_PAD - E
    src2d = jnp.concatenate([src, jnp.zeros((pad,), jnp.int32)]).reshape(-1, CH)
    dst2d = jnp.concatenate([dst, jnp.full((pad,), N, jnp.int32)]).reshape(-1, CH)

    h = x
    for i in range(5):
        agg = _seg_sum(h, src2d, dst2d)
        h = _mlp(h, agg, Wa[i], ba[i], Wb[i], bb[i])

    h_pad = jnp.concatenate([h, jnp.zeros((N_POOL_PAD - N, D), jnp.float32)])
    batch_pad = jnp.concatenate(
        [batch, jnp.full((N_POOL_PAD - N,), G, jnp.int32)])
    p = _pool(h_pad, batch_pad)
    return _final(p, Wlin, blin)


# speedup vs baseline: 1.2899x; 1.2899x over previous
import functools

import jax
import jax.numpy as jnp
from jax import lax
from jax.experimental import pallas as pl
from jax.experimental.pallas import tpu as pltpu
from jax.experimental.pallas import tpu_sc as plsc

N = 10000
D = 128
E = 320000
G = 256

NC = 2
NS = 16
NW = NC * NS

CH = 128
CPW = 80
E_PAD = NW * CPW * CH
N_ACC = 10240
SPC = 40
ZB = 32
OUT_STRIPE = 624

P_ROWS = 320
N_POOL_PAD = NW * P_ROWS
G_ACC = G + 8


def _seg_sum_body(h_hbm, src_hbm, dst_hbm, out_hbm,
                  src_v, dst_v, rows_v, zb_v, agg_sh, sem, sem1):
    cid = lax.axis_index("c")
    sid = lax.axis_index("s")
    wid = sid * NC + cid

    def zfill(i, _):
        for j in range(D // 16):
            zb_v[i, pl.ds(j * 16, 16)] = jnp.zeros((16,), jnp.float32)
        return 0
    lax.fori_loop(0, ZB, zfill, 0)

    def zcopy(m, _):
        pltpu.sync_copy(zb_v, agg_sh.at[pl.ds(sid * (N_ACC // NS) + m * ZB, ZB)])
        return 0
    lax.fori_loop(0, N_ACC // NS // ZB, zcopy, 0)

    plsc.subcore_barrier()

    rows0 = rows_v.at[0]
    rows1 = rows_v.at[1]
    for s in range(CPW // SPC):
        base = wid * CPW + s * SPC
        pltpu.sync_copy(src_hbm.at[pl.ds(base, SPC)], src_v)
        pltpu.sync_copy(dst_hbm.at[pl.ds(base, SPC)], dst_v)
        pltpu.async_copy(h_hbm.at[src_v.at[0]], rows0, sem)

        def pair(p, _):
            c0 = 2 * p
            pltpu.async_copy(h_hbm.at[src_v.at[c0 + 1]], rows1, sem1)
            pltpu.make_async_copy(h_hbm.at[src_v.at[c0]], rows0, sem).wait()
            pltpu.sync_copy(rows0, agg_sh.at[dst_v.at[c0]], add=True)
            pltpu.async_copy(h_hbm.at[src_v.at[c0 + 2]], rows0, sem)
            pltpu.make_async_copy(h_hbm.at[src_v.at[c0 + 1]], rows1, sem1).wait()
            pltpu.sync_copy(rows1, agg_sh.at[dst_v.at[c0 + 1]], add=True)
            return 0
        lax.fori_loop(0, SPC // 2 - 1, pair, 0)

        cl = SPC - 2
        pltpu.async_copy(h_hbm.at[src_v.at[cl + 1]], rows1, sem1)
        pltpu.make_async_copy(h_hbm.at[src_v.at[cl]], rows0, sem).wait()
        pltpu.sync_copy(rows0, agg_sh.at[dst_v.at[cl]], add=True)
        pltpu.make_async_copy(h_hbm.at[src_v.at[cl + 1]], rows1, sem1).wait()
        pltpu.sync_copy(rows1, agg_sh.at[dst_v.at[cl + 1]], add=True)

    plsc.subcore_barrier()

    @pl.when(sid < NS - 1)
    def _():
        pltpu.sync_copy(
            agg_sh.at[pl.ds(sid * OUT_STRIPE, OUT_STRIPE)],
            out_hbm.at[cid, pl.ds(sid * OUT_STRIPE, OUT_STRIPE)])

    @pl.when(sid == NS - 1)
    def _():
        last = (NS - 1) * OUT_STRIPE
        pltpu.sync_copy(
            agg_sh.at[pl.ds(last, N - last)],
            out_hbm.at[cid, pl.ds(last, N - last)])


@jax.jit
def _seg_sum(h, src2d, dst2d):
    mesh = plsc.VectorSubcoreMesh(core_axis_name="c", subcore_axis_name="s")
    return pl.kernel(
        _seg_sum_body,
        out_type=jax.ShapeDtypeStruct((NC, N, D), jnp.float32),
        mesh=mesh,
        scratch_types=[
            pltpu.VMEM((SPC, CH), jnp.int32),
            pltpu.VMEM((SPC, CH), jnp.int32),
            pltpu.VMEM((2, CH, D), jnp.float32),
            pltpu.VMEM((ZB, D), jnp.float32),
            pltpu.VMEM_SHARED((N_ACC, D), jnp.float32),
            pltpu.SemaphoreType.DMA,
            pltpu.SemaphoreType.DMA,
        ],
    )(h, src2d, dst2d)


def _mlp_body(h_ref, a_ref, wa_ref, ba_ref, wb_ref, bb_ref, o_ref):
    z = h_ref[...] + a_ref[0] + a_ref[1]
    z = jnp.dot(z.astype(jnp.bfloat16), wa_ref[...].astype(jnp.bfloat16),
                preferred_element_type=jnp.float32) + ba_ref[...]
    z = jnp.where(z > 0, z, jnp.exp(z) - 1.0)
    z = jnp.dot(z.astype(jnp.bfloat16), wb_ref[...].astype(jnp.bfloat16),
                preferred_element_type=jnp.float32) + bb_ref[...]
    o_ref[...] = jnp.where(z > 0, z, jnp.exp(z) - 1.0)


_MLP_BLK = 1000


@jax.jit
def _mlp(h, agg, wa, ba, wb, bb):
    return pl.pallas_call(
        _mlp_body,
        grid=(N // _MLP_BLK,),
        in_specs=[
            pl.BlockSpec((_MLP_BLK, D), lambda i: (i, 0)),
            pl.BlockSpec((NC, _MLP_BLK, D), lambda i: (0, i, 0)),
            pl.BlockSpec((D, D), lambda i: (0, 0)),
            pl.BlockSpec((1, D), lambda i: (0, 0)),
            pl.BlockSpec((D, D), lambda i: (0, 0)),
            pl.BlockSpec((1, D), lambda i: (0, 0)),
        ],
        out_specs=pl.BlockSpec((_MLP_BLK, D), lambda i: (i, 0)),
        out_shape=jax.ShapeDtypeStruct((N, D), jnp.float32),
    )(h, agg, wa, ba.reshape(1, D), wb, bb.reshape(1, D))


def _pool_body(h_hbm, batch_hbm, out_hbm, hv, bv, pool_v, tmp_v, acc_v,
               pool_sh, sem):
    cid = lax.axis_index("c")
    sid = lax.axis_index("s")
    wid = sid * NC + cid

    pltpu.sync_copy(h_hbm.at[pl.ds(wid * P_ROWS, P_ROWS)], hv)
    pltpu.sync_copy(batch_hbm.at[pl.ds(wid * P_ROWS, P_ROWS)], bv)

    ninf = jnp.full((16,), -jnp.inf, jnp.float32)

    def pinit(i, _):
        for j in range(D // 16):
            pool_v[i, pl.ds(j * 16, 16)] = ninf
        return 0
    lax.fori_loop(0, G_ACC, pinit, 0)

    def grp(g, _):
        b16 = bv[pl.ds(g * 16, 16)]
        for k in range(16):
            b = b16[k]
            i = g * 16 + k
            for j in range(D // 16):
                s = pl.ds(j * 16, 16)
                pool_v[b, s] = jnp.maximum(pool_v[b, s], hv[i, s])
        return 0
    lax.fori_loop(0, P_ROWS // 16, grp, 0)

    pltpu.sync_copy(pool_v.at[pl.ds(0, G)], pool_sh.at[sid])
    plsc.subcore_barrier()

    def ainit(i, _):
        for j in range(D // 16):
            acc_v[i, pl.ds(j * 16, 16)] = ninf
        return 0
    lax.fori_loop(0, 16, ainit, 0)

    def merge(t, _):
        pltpu.sync_copy(pool_sh.at[t, pl.ds(sid * 16, 16)], tmp_v)

        def mrow(i, _):
            for j in range(D // 16):
                s = pl.ds(j * 16, 16)
                acc_v[i, s] = jnp.maximum(acc_v[i, s], tmp_v[i, s])
            return 0
        lax.fori_loop(0, 16, mrow, 0)
        return 0
    lax.fori_loop(0, NS, merge, 0)

    pltpu.sync_copy(acc_v, out_hbm.at[cid, pl.ds(sid * 16, 16)])


@jax.jit
def _pool(h_pad, batch_pad):
    mesh = plsc.VectorSubcoreMesh(core_axis_name="c", subcore_axis_name="s")
    return pl.kernel(
        _pool_body,
        out_type=jax.ShapeDtypeStruct((NC, G, D), jnp.float32),
        mesh=mesh,
        scratch_types=[
            pltpu.VMEM((P_ROWS, D), jnp.float32),
            pltpu.VMEM((P_ROWS,), jnp.int32),
            pltpu.VMEM((G_ACC, D), jnp.float32),
            pltpu.VMEM((16, D), jnp.float32),
            pltpu.VMEM((16, D), jnp.float32),
            pltpu.VMEM_SHARED((NS, G, D), jnp.float32),
            pltpu.SemaphoreType.DMA,
        ],
    )(h_pad, batch_pad)


def _final_body(p_ref, w_ref, b_ref, o_ref):
    pooled = jnp.maximum(p_ref[0], p_ref[1])
    o_ref[...] = jnp.dot(pooled.astype(jnp.bfloat16),
                         w_ref[...].astype(jnp.bfloat16),
                         preferred_element_type=jnp.float32) + b_ref[...]


@jax.jit
def _final(p, wlin, blin):
    c = wlin.shape[1]
    return pl.pallas_call(
        _final_body,
        out_shape=jax.ShapeDtypeStruct((G, c), jnp.float32),
    )(p, wlin, blin.reshape(1, c))


def kernel(x, edge_index, batch, Wa, ba, Wb, bb, Wlin, blin):
    src = edge_index[0]
    dst = edge_index[1]
    pad = E_PAD - E
    src2d = jnp.concatenate([src, jnp.zeros((pad,), jnp.int32)]).reshape(-1, CH)
    dst2d = jnp.concatenate([dst, jnp.full((pad,), N, jnp.int32)]).reshape(-1, CH)

    h = x
    for i in range(5):
        agg = _seg_sum(h, src2d, dst2d)
        h = _mlp(h, agg, Wa[i], ba[i], Wb[i], bb[i])

    h_pad = jnp.concatenate([h, jnp.zeros((N_POOL_PAD - N, D), jnp.float32)])
    batch_pad = jnp.concatenate(
        [batch, jnp.full((N_POOL_PAD - N,), G, jnp.int32)])
    p = _pool(h_pad, batch_pad)
    return _final(p, Wlin, blin)
